# baseline (speedup 1.0000x reference)
import jax
import jax.numpy as jnp
from jax import lax
from jax.experimental import pallas as pl
from jax.experimental.pallas import tpu as pltpu

B, J, D, W = 8, 2048, 128, 4
H_H, H_E, LATENT = 256, 256, 32
JB = 256
NJ = J // JB
R = B * JB


def _ln(v, g, b, eps=1e-5):
    mu = jnp.mean(v, axis=-1, keepdims=True)
    xc = v - mu
    var = jnp.mean(xc * xc, axis=-1, keepdims=True)
    return xc * lax.rsqrt(var + eps) * g + b


def _body(xT, M32, femb, w1x, w1f, b1, ln1g, ln1b, W2, b2, ln2g, ln2b,
          gW1, gb1, gW2P, gb2P, cW, cb, clng, clnb,
          eW1, eb1, eln1g, eln1b, eW2, eb2, eln2g, eln2b,
          mu_out, lv_out, acc, den):
    j = pl.program_id(0)

    @pl.when(j == 0)
    def _init():
        acc[:] = jnp.zeros_like(acc[:])
        den[:] = jnp.zeros_like(den[:])

    F1 = jnp.dot(femb[:], w1f[:], preferred_element_type=jnp.float32) + b1[:]
    xblk = xT[:]

    h1 = jnp.concatenate(
        [F1 + xblk[:, b:b + 1] * w1x[:] for b in range(B)], axis=0)
    h1 = jnp.maximum(_ln(h1, ln1g[:], ln1b[:]), 0.0)
    h2 = jnp.dot(h1, W2[:], preferred_element_type=jnp.float32) + b2[:]
    h = jnp.maximum(_ln(h2, ln2g[:], ln2b[:]), 0.0)
    g1 = jnp.maximum(
        jnp.dot(h, gW1[:], preferred_element_type=jnp.float32) + gb1[:], 0.0)
    raw32 = jnp.dot(g1, gW2P[:], preferred_element_type=jnp.float32) + gb2P[:]
    E = M32[0] * jnp.exp(jnp.clip(raw32, -10.0, 10.0))

    dn = (((0,), (0,)), ((), ()))
    acc[:] += lax.dot_general(E, h, dn, preferred_element_type=jnp.float32)
    den[:] += lax.dot_general(E, jnp.ones((R, D), jnp.float32), dn,
                              preferred_element_type=jnp.float32)

    @pl.when(j == NJ - 1)
    def _final():
        c = cb[:]
        for w in range(W):
            dw = den[w * B:(w + 1) * B, :]
            hw = jnp.where(dw > 0, acc[w * B:(w + 1) * B, :] / dw, 0.0)
            c = c + jnp.dot(hw, cW[w * D:(w + 1) * D, :],
                            preferred_element_type=jnp.float32)
        c = jnp.maximum(_ln(c, clng[:], clnb[:]), 0.0)
        has = den[0:B, 0:1] > 0
        c = jnp.where(has, c, 0.0)
        e1 = jnp.dot(c, eW1[:], preferred_element_type=jnp.float32) + eb1[:]
        e1 = jnp.maximum(_ln(e1, eln1g[:], eln1b[:]), 0.0)
        e2 = jnp.dot(e1, eW2[:], preferred_element_type=jnp.float32) + eb2[:]
        e2 = jnp.maximum(_ln(e2, eln2g[:], eln2b[:]), 0.0)
        mu_out[:] = e2[:, :LATENT]
        lv_out[:] = e2[:, LATENT:]


def _full(shape):
    return pl.BlockSpec(shape, lambda j: tuple(0 for _ in shape))


def kernel(x, mask, params, interpret=False):
    p = params
    xT = x.T
    row = lambda a: a[None, :]

    m1 = mask.reshape(B, NJ, JB).transpose(1, 0, 2).reshape(NJ, R)
    ind = (jnp.arange(R)[:, None] // JB == jnp.arange(W * B)[None, :] % B)
    M32 = ((m1[:, :, None] > 0) & ind[None]).astype(jnp.float32)

    P = jnp.repeat(jnp.eye(W, dtype=jnp.float32), B, axis=1)
    gW2P = p["g_W2"] @ P
    gb2P = row(p["g_b2"] @ P)

    in_arrays = [
        xT, M32, p["feature_embedding"],
        row(p["h_W1"][0]), p["h_W1"][1:], row(p["h_b1"]),
        row(p["h_ln1_g"]), row(p["h_ln1_b"]),
        p["h_W2"], row(p["h_b2"]), row(p["h_ln2_g"]), row(p["h_ln2_b"]),
        p["g_W1"], row(p["g_b1"]), gW2P, gb2P,
        p["c_W"], row(p["c_b"]), row(p["c_ln_g"]), row(p["c_ln_b"]),
        p["e_W1"], row(p["e_b1"]), row(p["e_ln1_g"]), row(p["e_ln1_b"]),
        p["e_W2"], row(p["e_b2"]), row(p["e_ln2_g"]), row(p["e_ln2_b"]),
    ]
    in_specs = [
        pl.BlockSpec((JB, B), lambda j: (j, 0)),
        pl.BlockSpec((1, R, W * B), lambda j: (j, 0, 0)),
        pl.BlockSpec((JB, D), lambda j: (j, 0)),
    ] + [_full(a.shape) for a in in_arrays[3:]]

    mu, lv = pl.pallas_call(
        _body,
        grid=(NJ,),
        in_specs=in_specs,
        out_specs=[_full((B, LATENT)), _full((B, LATENT))],
        out_shape=[jax.ShapeDtypeStruct((B, LATENT), jnp.float32),
                   jax.ShapeDtypeStruct((B, LATENT), jnp.float32)],
        scratch_shapes=[pltpu.VMEM((W * B, D), jnp.float32),
                        pltpu.VMEM((W * B, D), jnp.float32)],
        compiler_params=pltpu.CompilerParams(
            dimension_semantics=("arbitrary",)),
        interpret=interpret,
    )(*in_arrays)
    return (mu, lv)


# speedup vs baseline: 1.0332x; 1.0332x over previous
import jax
import jax.numpy as jnp
from jax import lax
from jax.experimental import pallas as pl
from jax.experimental.pallas import tpu as pltpu

B, J, D, W = 8, 2048, 128, 4
H_H, H_E, LATENT = 256, 256, 32
JB = 256
NJ = J // JB
R = B * JB


def _ln(v, g, b, eps=1e-5):
    mu = jnp.mean(v, axis=-1, keepdims=True)
    xc = v - mu
    var = jnp.mean(xc * xc, axis=-1, keepdims=True)
    return xc * lax.rsqrt(var + eps) * g + b


def _body(xT, mT, ind32, femb, w1x, w1f, b1, ln1g, ln1b, W2, b2, ln2g, ln2b,
          gW1, gb1, gW2P, gb2P, cW, cb, clng, clnb,
          eW1, eb1, eln1g, eln1b, eW2, eb2, eln2g, eln2b,
          mu_out, lv_out, acc, den):
    j = pl.program_id(0)

    @pl.when(j == 0)
    def _init():
        acc[:] = jnp.zeros_like(acc[:])
        den[:] = jnp.zeros_like(den[:])

    F1 = jnp.dot(femb[:], w1f[:], preferred_element_type=jnp.float32) + b1[:]
    xblk = xT[:]
    mblk = mT[:]

    h1 = jnp.concatenate(
        [F1 + xblk[:, b:b + 1] * w1x[:] for b in range(B)], axis=0)
    h1 = jnp.maximum(_ln(h1, ln1g[:], ln1b[:]), 0.0)
    h2 = jnp.dot(h1, W2[:], preferred_element_type=jnp.float32) + b2[:]
    h = jnp.maximum(_ln(h2, ln2g[:], ln2b[:]), 0.0)
    g1 = jnp.maximum(
        jnp.dot(h, gW1[:], preferred_element_type=jnp.float32) + gb1[:], 0.0)
    raw32 = jnp.dot(g1, gW2P[:], preferred_element_type=jnp.float32) + gb2P[:]
    msel = jnp.concatenate(
        [mblk[:, b:b + 1] for b in range(B)], axis=0)
    E = ind32[:] * (msel * jnp.exp(jnp.clip(raw32, -10.0, 10.0)))

    dn = (((0,), (0,)), ((), ()))
    acc[:] += lax.dot_general(E, h, dn, preferred_element_type=jnp.float32)
    den[:] += lax.dot_general(E, jnp.ones((R, D), jnp.float32), dn,
                              preferred_element_type=jnp.float32)

    @pl.when(j == NJ - 1)
    def _final():
        c = cb[:]
        for w in range(W):
            dw = den[w * B:(w + 1) * B, :]
            hw = jnp.where(dw > 0, acc[w * B:(w + 1) * B, :] / dw, 0.0)
            c = c + jnp.dot(hw, cW[w * D:(w + 1) * D, :],
                            preferred_element_type=jnp.float32)
        c = jnp.maximum(_ln(c, clng[:], clnb[:]), 0.0)
        has = den[0:B, 0:1] > 0
        c = jnp.where(has, c, 0.0)
        e1 = jnp.dot(c, eW1[:], preferred_element_type=jnp.float32) + eb1[:]
        e1 = jnp.maximum(_ln(e1, eln1g[:], eln1b[:]), 0.0)
        e2 = jnp.dot(e1, eW2[:], preferred_element_type=jnp.float32) + eb2[:]
        e2 = jnp.maximum(_ln(e2, eln2g[:], eln2b[:]), 0.0)
        mu_out[:] = e2[:, :LATENT]
        lv_out[:] = e2[:, LATENT:]


def _full(shape):
    return pl.BlockSpec(shape, lambda j: tuple(0 for _ in shape))


def kernel(x, mask, params, interpret=False):
    p = params
    xT = x.T
    row = lambda a: a[None, :]

    mT = mask.astype(jnp.float32).T
    ind32 = (jnp.arange(R)[:, None] // JB
             == jnp.arange(W * B)[None, :] % B).astype(jnp.float32)

    P = jnp.repeat(jnp.eye(W, dtype=jnp.float32), B, axis=1)
    gW2P = p["g_W2"] @ P
    gb2P = row(p["g_b2"] @ P)

    in_arrays = [
        xT, mT, ind32, p["feature_embedding"],
        row(p["h_W1"][0]), p["h_W1"][1:], row(p["h_b1"]),
        row(p["h_ln1_g"]), row(p["h_ln1_b"]),
        p["h_W2"], row(p["h_b2"]), row(p["h_ln2_g"]), row(p["h_ln2_b"]),
        p["g_W1"], row(p["g_b1"]), gW2P, gb2P,
        p["c_W"], row(p["c_b"]), row(p["c_ln_g"]), row(p["c_ln_b"]),
        p["e_W1"], row(p["e_b1"]), row(p["e_ln1_g"]), row(p["e_ln1_b"]),
        p["e_W2"], row(p["e_b2"]), row(p["e_ln2_g"]), row(p["e_ln2_b"]),
    ]
    in_specs = [
        pl.BlockSpec((JB, B), lambda j: (j, 0)),
        pl.BlockSpec((JB, B), lambda j: (j, 0)),
        _full((R, W * B)),
        pl.BlockSpec((JB, D), lambda j: (j, 0)),
    ] + [_full(a.shape) for a in in_arrays[4:]]

    mu, lv = pl.pallas_call(
        _body,
        grid=(NJ,),
        in_specs=in_specs,
        out_specs=[_full((B, LATENT)), _full((B, LATENT))],
        out_shape=[jax.ShapeDtypeStruct((B, LATENT), jnp.float32),
                   jax.ShapeDtypeStruct((B, LATENT), jnp.float32)],
        scratch_shapes=[pltpu.VMEM((W * B, D), jnp.float32),
                        pltpu.VMEM((W * B, D), jnp.float32)],
        compiler_params=pltpu.CompilerParams(
            dimension_semantics=("arbitrary",)),
        interpret=interpret,
    )(*in_arrays)
    return (mu, lv)


# speedup vs baseline: 1.2468x; 1.2068x over previous
import jax
import jax.numpy as jnp
from jax import lax
from jax.experimental import pallas as pl
from jax.experimental.pallas import tpu as pltpu

B, J, D, W = 8, 2048, 128, 4
H_H, H_E, LATENT = 256, 256, 32
JB = 256
NJ = J // JB
R = B * JB


def _ln(v, g, b, eps=1e-5):
    mu = jnp.mean(v, axis=-1, keepdims=True)
    xc = v - mu
    var = jnp.mean(xc * xc, axis=-1, keepdims=True)
    return xc * lax.rsqrt(var + eps) * g + b


def _body(x_ref, m_ref, femb, hW1, b1, ln1g, ln1b, W2, b2, ln2g, ln2b,
          gW1, gb1, gW2, gb2, cW, cb, clng, clnb,
          eW1, eb1, eln1g, eln1b, eW2, eb2, eln2g, eln2b,
          mu_out, lv_out, acc, den, ind32, gW2P):
    j = pl.program_id(0)
    w1x = hW1[0:1, :]
    w1f = hW1[1:, :]

    @pl.when(j == 0)
    def _init():
        acc[:] = jnp.zeros_like(acc[:])
        den[:] = jnp.zeros_like(den[:])
        ind32[:] = (lax.broadcasted_iota(jnp.int32, (R, W * B), 0) // JB
                    == lax.broadcasted_iota(jnp.int32, (R, W * B), 1) % B
                    ).astype(jnp.float32)
        P = (lax.broadcasted_iota(jnp.int32, (W, W * B), 0)
             == lax.broadcasted_iota(jnp.int32, (W, W * B), 1) // B
             ).astype(jnp.float32)
        gW2P[:] = jnp.dot(gW2[:], P, preferred_element_type=jnp.float32)

    F1 = jnp.dot(femb[:], w1f, preferred_element_type=jnp.float32) + b1[:]
    xblk = x_ref[:].T
    mblk = m_ref[:].astype(jnp.float32).T

    h1 = jnp.concatenate(
        [F1 + xblk[:, b:b + 1] * w1x for b in range(B)], axis=0)
    h1 = jnp.maximum(_ln(h1, ln1g[:], ln1b[:]), 0.0)
    h2 = jnp.dot(h1, W2[:], preferred_element_type=jnp.float32) + b2[:]
    h = jnp.maximum(_ln(h2, ln2g[:], ln2b[:]), 0.0)
    g1 = jnp.maximum(
        jnp.dot(h, gW1[:], preferred_element_type=jnp.float32) + gb1[:], 0.0)
    raw32 = (jnp.dot(g1, gW2P[:], preferred_element_type=jnp.float32)
             + jnp.repeat(gb2[:], B, axis=1))
    msel = jnp.concatenate(
        [mblk[:, b:b + 1] for b in range(B)], axis=0)
    E = ind32[:] * (msel * jnp.exp(jnp.clip(raw32, -10.0, 10.0)))

    dn = (((0,), (0,)), ((), ()))
    acc[:] += lax.dot_general(E, h, dn, preferred_element_type=jnp.float32)
    den[:] += lax.dot_general(E, jnp.ones((R, D), jnp.float32), dn,
                              preferred_element_type=jnp.float32)

    @pl.when(j == NJ - 1)
    def _final():
        c = cb[:]
        for w in range(W):
            dw = den[w * B:(w + 1) * B, :]
            hw = jnp.where(dw > 0, acc[w * B:(w + 1) * B, :] / dw, 0.0)
            c = c + jnp.dot(hw, cW[w * D:(w + 1) * D, :],
                            preferred_element_type=jnp.float32)
        c = jnp.maximum(_ln(c, clng[:], clnb[:]), 0.0)
        has = den[0:B, 0:1] > 0
        c = jnp.where(has, c, 0.0)
        e1 = jnp.dot(c, eW1[:], preferred_element_type=jnp.float32) + eb1[:]
        e1 = jnp.maximum(_ln(e1, eln1g[:], eln1b[:]), 0.0)
        e2 = jnp.dot(e1, eW2[:], preferred_element_type=jnp.float32) + eb2[:]
        e2 = jnp.maximum(_ln(e2, eln2g[:], eln2b[:]), 0.0)
        mu_out[:] = e2[:, :LATENT]
        lv_out[:] = e2[:, LATENT:]


def _full(shape):
    return pl.BlockSpec(shape, lambda j: tuple(0 for _ in shape))


def kernel(x, mask, params, interpret=False):
    p = params
    row = lambda a: a[None, :]

    in_arrays = [
        x, mask, p["feature_embedding"],
        p["h_W1"], row(p["h_b1"]),
        row(p["h_ln1_g"]), row(p["h_ln1_b"]),
        p["h_W2"], row(p["h_b2"]), row(p["h_ln2_g"]), row(p["h_ln2_b"]),
        p["g_W1"], row(p["g_b1"]), p["g_W2"], row(p["g_b2"]),
        p["c_W"], row(p["c_b"]), row(p["c_ln_g"]), row(p["c_ln_b"]),
        p["e_W1"], row(p["e_b1"]), row(p["e_ln1_g"]), row(p["e_ln1_b"]),
        p["e_W2"], row(p["e_b2"]), row(p["e_ln2_g"]), row(p["e_ln2_b"]),
    ]
    in_specs = [
        pl.BlockSpec((B, JB), lambda j: (0, j)),
        pl.BlockSpec((B, JB), lambda j: (0, j)),
        pl.BlockSpec((JB, D), lambda j: (j, 0)),
    ] + [_full(a.shape) for a in in_arrays[3:]]

    mu, lv = pl.pallas_call(
        _body,
        grid=(NJ,),
        in_specs=in_specs,
        out_specs=[_full((B, LATENT)), _full((B, LATENT))],
        out_shape=[jax.ShapeDtypeStruct((B, LATENT), jnp.float32),
                   jax.ShapeDtypeStruct((B, LATENT), jnp.float32)],
        scratch_shapes=[pltpu.VMEM((W * B, D), jnp.float32),
                        pltpu.VMEM((W * B, D), jnp.float32),
                        pltpu.VMEM((R, W * B), jnp.float32),
                        pltpu.VMEM((p["g_W2"].shape[0], W * B), jnp.float32)],
        compiler_params=pltpu.CompilerParams(
            dimension_semantics=("arbitrary",)),
        interpret=interpret,
    )(*in_arrays)
    return (mu, lv)


# speedup vs baseline: 1.3679x; 1.0971x over previous
import jax
import jax.numpy as jnp
from jax import lax
from jax.experimental import pallas as pl
from jax.experimental.pallas import tpu as pltpu

B, J, D, W = 8, 2048, 128, 4
H_H, H_E, LATENT = 256, 256, 32
JB = 1024
NJ = J // JB
R = B * JB


def _ln(v, g, b, eps=1e-5):
    mu = jnp.mean(v, axis=-1, keepdims=True)
    xc = v - mu
    var = jnp.mean(xc * xc, axis=-1, keepdims=True)
    return xc * lax.rsqrt(var + eps) * g + b


def _body(x_ref, m_ref, femb, hW1, b1, ln1g, ln1b, W2, b2, ln2g, ln2b,
          gW1, gb1, gW2, gb2, cW, cb, clng, clnb,
          eW1, eb1, eln1g, eln1b, eW2, eb2, eln2g, eln2b,
          mu_out, lv_out, acc, den, ind32, gW2P):
    j = pl.program_id(0)
    w1x = hW1[0:1, :]
    w1f = hW1[1:, :]

    @pl.when(j == 0)
    def _init():
        acc[:] = jnp.zeros_like(acc[:])
        den[:] = jnp.zeros_like(den[:])
        ind32[:] = (lax.broadcasted_iota(jnp.int32, (R, W * B), 0) // JB
                    == lax.broadcasted_iota(jnp.int32, (R, W * B), 1) % B
                    ).astype(jnp.float32)
        P = (lax.broadcasted_iota(jnp.int32, (W, W * B), 0)
             == lax.broadcasted_iota(jnp.int32, (W, W * B), 1) // B
             ).astype(jnp.float32)
        gW2P[:] = jnp.dot(gW2[:], P, preferred_element_type=jnp.float32)

    F1 = jnp.dot(femb[:], w1f, preferred_element_type=jnp.float32) + b1[:]
    xblk = x_ref[:].T
    mblk = m_ref[:].astype(jnp.float32).T

    h1 = jnp.concatenate(
        [F1 + xblk[:, b:b + 1] * w1x for b in range(B)], axis=0)
    h1 = jnp.maximum(_ln(h1, ln1g[:], ln1b[:]), 0.0)
    h2 = jnp.dot(h1, W2[:], preferred_element_type=jnp.float32) + b2[:]
    h = jnp.maximum(_ln(h2, ln2g[:], ln2b[:]), 0.0)
    g1 = jnp.maximum(
        jnp.dot(h, gW1[:], preferred_element_type=jnp.float32) + gb1[:], 0.0)
    raw32 = (jnp.dot(g1, gW2P[:], preferred_element_type=jnp.float32)
             + jnp.repeat(gb2[:], B, axis=1))
    msel = jnp.concatenate(
        [mblk[:, b:b + 1] for b in range(B)], axis=0)
    E = ind32[:] * (msel * jnp.exp(jnp.clip(raw32, -10.0, 10.0)))

    dn = (((0,), (0,)), ((), ()))
    acc[:] += lax.dot_general(E, h, dn, preferred_element_type=jnp.float32)
    den[:] += lax.dot_general(E, jnp.ones((R, D), jnp.float32), dn,
                              preferred_element_type=jnp.float32)

    @pl.when(j == NJ - 1)
    def _final():
        c = cb[:]
        for w in range(W):
            dw = den[w * B:(w + 1) * B, :]
            hw = jnp.where(dw > 0, acc[w * B:(w + 1) * B, :] / dw, 0.0)
            c = c + jnp.dot(hw, cW[w * D:(w + 1) * D, :],
                            preferred_element_type=jnp.float32)
        c = jnp.maximum(_ln(c, clng[:], clnb[:]), 0.0)
        has = den[0:B, 0:1] > 0
        c = jnp.where(has, c, 0.0)
        e1 = jnp.dot(c, eW1[:], preferred_element_type=jnp.float32) + eb1[:]
        e1 = jnp.maximum(_ln(e1, eln1g[:], eln1b[:]), 0.0)
        e2 = jnp.dot(e1, eW2[:], preferred_element_type=jnp.float32) + eb2[:]
        e2 = jnp.maximum(_ln(e2, eln2g[:], eln2b[:]), 0.0)
        mu_out[:] = e2[:, :LATENT]
        lv_out[:] = e2[:, LATENT:]


def _full(shape):
    return pl.BlockSpec(shape, lambda j: tuple(0 for _ in shape))


def kernel(x, mask, params, interpret=False):
    p = params
    row = lambda a: a[None, :]

    in_arrays = [
        x, mask, p["feature_embedding"],
        p["h_W1"], row(p["h_b1"]),
        row(p["h_ln1_g"]), row(p["h_ln1_b"]),
        p["h_W2"], row(p["h_b2"]), row(p["h_ln2_g"]), row(p["h_ln2_b"]),
        p["g_W1"], row(p["g_b1"]), p["g_W2"], row(p["g_b2"]),
        p["c_W"], row(p["c_b"]), row(p["c_ln_g"]), row(p["c_ln_b"]),
        p["e_W1"], row(p["e_b1"]), row(p["e_ln1_g"]), row(p["e_ln1_b"]),
        p["e_W2"], row(p["e_b2"]), row(p["e_ln2_g"]), row(p["e_ln2_b"]),
    ]
    in_specs = [
        pl.BlockSpec((B, JB), lambda j: (0, j)),
        pl.BlockSpec((B, JB), lambda j: (0, j)),
        pl.BlockSpec((JB, D), lambda j: (j, 0)),
    ] + [_full(a.shape) for a in in_arrays[3:]]

    mu, lv = pl.pallas_call(
        _body,
        grid=(NJ,),
        in_specs=in_specs,
        out_specs=[_full((B, LATENT)), _full((B, LATENT))],
        out_shape=[jax.ShapeDtypeStruct((B, LATENT), jnp.float32),
                   jax.ShapeDtypeStruct((B, LATENT), jnp.float32)],
        scratch_shapes=[pltpu.VMEM((W * B, D), jnp.float32),
                        pltpu.VMEM((W * B, D), jnp.float32),
                        pltpu.VMEM((R, W * B), jnp.float32),
                        pltpu.VMEM((p["g_W2"].shape[0], W * B), jnp.float32)],
        compiler_params=pltpu.CompilerParams(
            dimension_semantics=("arbitrary",)),
        interpret=interpret,
    )(*in_arrays)
    return (mu, lv)


# speedup vs baseline: 1.3802x; 1.0090x over previous
import jax
import jax.numpy as jnp
from jax import lax
from jax.experimental import pallas as pl
from jax.experimental.pallas import tpu as pltpu

B, J, D, W = 8, 2048, 128, 4
H_H, H_E, LATENT = 256, 256, 32
JB = 2048
NJ = J // JB
R = B * JB


def _ln(v, g, b, eps=1e-5):
    mu = jnp.mean(v, axis=-1, keepdims=True)
    xc = v - mu
    var = jnp.mean(xc * xc, axis=-1, keepdims=True)
    return xc * lax.rsqrt(var + eps) * g + b


def _body(x_ref, m_ref, femb, hW1, b1, ln1g, ln1b, W2, b2, ln2g, ln2b,
          gW1, gb1, gW2, gb2, cW, cb, clng, clnb,
          eW1, eb1, eln1g, eln1b, eW2, eb2, eln2g, eln2b,
          mu_out, lv_out, acc, den, ind32, gW2P):
    j = pl.program_id(0)
    w1x = hW1[0:1, :]
    w1f = hW1[1:, :]

    @pl.when(j == 0)
    def _init():
        acc[:] = jnp.zeros_like(acc[:])
        den[:] = jnp.zeros_like(den[:])
        ind32[:] = (lax.broadcasted_iota(jnp.int32, (R, W * B), 0) // JB
                    == lax.broadcasted_iota(jnp.int32, (R, W * B), 1) % B
                    ).astype(jnp.float32)
        P = (lax.broadcasted_iota(jnp.int32, (W, W * B), 0)
             == lax.broadcasted_iota(jnp.int32, (W, W * B), 1) // B
             ).astype(jnp.float32)
        gW2P[:] = jnp.dot(gW2[:], P, preferred_element_type=jnp.float32)

    F1 = jnp.dot(femb[:], w1f, preferred_element_type=jnp.float32) + b1[:]
    xblk = x_ref[:].T
    mblk = m_ref[:].astype(jnp.float32).T

    h1 = jnp.concatenate(
        [F1 + xblk[:, b:b + 1] * w1x for b in range(B)], axis=0)
    h1 = jnp.maximum(_ln(h1, ln1g[:], ln1b[:]), 0.0)
    h2 = jnp.dot(h1, W2[:], preferred_element_type=jnp.float32) + b2[:]
    h = jnp.maximum(_ln(h2, ln2g[:], ln2b[:]), 0.0)
    g1 = jnp.maximum(
        jnp.dot(h, gW1[:], preferred_element_type=jnp.float32) + gb1[:], 0.0)
    raw32 = (jnp.dot(g1, gW2P[:], preferred_element_type=jnp.float32)
             + jnp.repeat(gb2[:], B, axis=1))
    msel = jnp.concatenate(
        [mblk[:, b:b + 1] for b in range(B)], axis=0)
    E = ind32[:] * (msel * jnp.exp(jnp.clip(raw32, -10.0, 10.0)))

    dn = (((0,), (0,)), ((), ()))
    acc[:] += lax.dot_general(E, h, dn, preferred_element_type=jnp.float32)
    den[:] += lax.dot_general(E, jnp.ones((R, D), jnp.float32), dn,
                              preferred_element_type=jnp.float32)

    @pl.when(j == NJ - 1)
    def _final():
        c = cb[:]
        for w in range(W):
            dw = den[w * B:(w + 1) * B, :]
            hw = jnp.where(dw > 0, acc[w * B:(w + 1) * B, :] / dw, 0.0)
            c = c + jnp.dot(hw, cW[w * D:(w + 1) * D, :],
                            preferred_element_type=jnp.float32)
        c = jnp.maximum(_ln(c, clng[:], clnb[:]), 0.0)
        has = den[0:B, 0:1] > 0
        c = jnp.where(has, c, 0.0)
        e1 = jnp.dot(c, eW1[:], preferred_element_type=jnp.float32) + eb1[:]
        e1 = jnp.maximum(_ln(e1, eln1g[:], eln1b[:]), 0.0)
        e2 = jnp.dot(e1, eW2[:], preferred_element_type=jnp.float32) + eb2[:]
        e2 = jnp.maximum(_ln(e2, eln2g[:], eln2b[:]), 0.0)
        mu_out[:] = e2[:, :LATENT]
        lv_out[:] = e2[:, LATENT:]


def _full(shape):
    return pl.BlockSpec(shape, lambda j: tuple(0 for _ in shape))


def kernel(x, mask, params, interpret=False):
    p = params
    row = lambda a: a[None, :]

    in_arrays = [
        x, mask, p["feature_embedding"],
        p["h_W1"], row(p["h_b1"]),
        row(p["h_ln1_g"]), row(p["h_ln1_b"]),
        p["h_W2"], row(p["h_b2"]), row(p["h_ln2_g"]), row(p["h_ln2_b"]),
        p["g_W1"], row(p["g_b1"]), p["g_W2"], row(p["g_b2"]),
        p["c_W"], row(p["c_b"]), row(p["c_ln_g"]), row(p["c_ln_b"]),
        p["e_W1"], row(p["e_b1"]), row(p["e_ln1_g"]), row(p["e_ln1_b"]),
        p["e_W2"], row(p["e_b2"]), row(p["e_ln2_g"]), row(p["e_ln2_b"]),
    ]
    in_specs = [
        pl.BlockSpec((B, JB), lambda j: (0, j)),
        pl.BlockSpec((B, JB), lambda j: (0, j)),
        pl.BlockSpec((JB, D), lambda j: (j, 0)),
    ] + [_full(a.shape) for a in in_arrays[3:]]

    mu, lv = pl.pallas_call(
        _body,
        grid=(NJ,),
        in_specs=in_specs,
        out_specs=[_full((B, LATENT)), _full((B, LATENT))],
        out_shape=[jax.ShapeDtypeStruct((B, LATENT), jnp.float32),
                   jax.ShapeDtypeStruct((B, LATENT), jnp.float32)],
        scratch_shapes=[pltpu.VMEM((W * B, D), jnp.float32),
                        pltpu.VMEM((W * B, D), jnp.float32),
                        pltpu.VMEM((R, W * B), jnp.float32),
                        pltpu.VMEM((p["g_W2"].shape[0], W * B), jnp.float32)],
        compiler_params=pltpu.CompilerParams(
            dimension_semantics=("arbitrary",)),
        interpret=interpret,
    )(*in_arrays)
    return (mu, lv)


# speedup vs baseline: 1.3811x; 1.0007x over previous
import jax
import jax.numpy as jnp
from jax import lax
from jax.experimental import pallas as pl
from jax.experimental.pallas import tpu as pltpu

B, J, D, W = 8, 2048, 128, 4
H_H, H_E, LATENT = 256, 256, 32
JB = 2048
NJ = J // JB
R = B * JB


def _ln(v, g, b, eps=1e-5):
    mu = jnp.mean(v, axis=-1, keepdims=True)
    xc = v - mu
    var = jnp.mean(xc * xc, axis=-1, keepdims=True)
    return xc * lax.rsqrt(var + eps) * g + b


def _body(x_ref, m_ref, femb, hW1, b1, ln1g, ln1b, W2, b2, ln2g, ln2b,
          gW1, gb1, gW2, gb2, cW, cb, clng, clnb,
          eW1, eb1, eln1g, eln1b, eW2, eb2, eln2g, eln2b,
          mu_out, lv_out, acc, den, ind32, gW2P):
    j = pl.program_id(0)
    w1x = hW1[0:1, :]
    w1f = hW1[1:, :]

    @pl.when(j == 0)
    def _init():
        acc[:] = jnp.zeros_like(acc[:])
        den[:] = jnp.zeros_like(den[:])
        ind32[:] = (lax.broadcasted_iota(jnp.int32, (R, W * B), 0) // JB
                    == lax.broadcasted_iota(jnp.int32, (R, W * B), 1) % B
                    ).astype(jnp.float32)
        P = (lax.broadcasted_iota(jnp.int32, (W, W * B), 0)
             == lax.broadcasted_iota(jnp.int32, (W, W * B), 1) // B
             ).astype(jnp.float32)
        gW2P[:] = jnp.dot(gW2[:], P, preferred_element_type=jnp.float32)

    F1 = jnp.dot(femb[:], w1f, preferred_element_type=jnp.float32) + b1[:]
    xblk = x_ref[:].T
    mblk = m_ref[:].astype(jnp.float32).T

    h1 = jnp.concatenate(
        [F1 + xblk[:, b:b + 1] * w1x for b in range(B)], axis=0)
    h1 = jnp.maximum(_ln(h1, ln1g[:], ln1b[:]), 0.0)
    h2 = jnp.dot(h1, W2[:], preferred_element_type=jnp.float32) + b2[:]
    h = jnp.maximum(_ln(h2, ln2g[:], ln2b[:]), 0.0)
    g1 = jnp.maximum(
        jnp.dot(h, gW1[:], preferred_element_type=jnp.float32) + gb1[:], 0.0)
    raw32 = (jnp.dot(g1, gW2P[:], preferred_element_type=jnp.float32)
             + jnp.repeat(gb2[:], B, axis=1))
    msel = jnp.concatenate(
        [mblk[:, b:b + 1] for b in range(B)], axis=0)
    E = ind32[:] * (msel * jnp.exp(jnp.clip(raw32, -10.0, 10.0)))

    dn = (((0,), (0,)), ((), ()))
    acc[:] += lax.dot_general(E, h, dn, preferred_element_type=jnp.float32)
    den[:] += lax.dot_general(E, jnp.ones((R, D), jnp.float32), dn,
                              preferred_element_type=jnp.float32)

    @pl.when(j == NJ - 1)
    def _final():
        c = cb[:]
        for w in range(W):
            dw = den[w * B:(w + 1) * B, :]
            hw = jnp.where(dw > 0, acc[w * B:(w + 1) * B, :] / dw, 0.0)
            c = c + jnp.dot(hw, cW[w * D:(w + 1) * D, :],
                            preferred_element_type=jnp.float32)
        c = jnp.maximum(_ln(c, clng[:], clnb[:]), 0.0)
        has = den[0:B, 0:1] > 0
        c = jnp.where(has, c, 0.0)
        e1 = jnp.dot(c, eW1[:], preferred_element_type=jnp.float32) + eb1[:]
        e1 = jnp.maximum(_ln(e1, eln1g[:], eln1b[:]), 0.0)
        e2 = jnp.dot(e1, eW2[:], preferred_element_type=jnp.float32) + eb2[:]
        e2 = jnp.maximum(_ln(e2, eln2g[:], eln2b[:]), 0.0)
        mu_out[:] = e2[:, :LATENT]
        lv_out[:] = e2[:, LATENT:]


def _full(shape):
    return pl.BlockSpec(shape, lambda j: tuple(0 for _ in shape))


def kernel(x, mask, params):
    p = params
    row = lambda a: a[None, :]

    in_arrays = [
        x, mask, p["feature_embedding"],
        p["h_W1"], row(p["h_b1"]),
        row(p["h_ln1_g"]), row(p["h_ln1_b"]),
        p["h_W2"], row(p["h_b2"]), row(p["h_ln2_g"]), row(p["h_ln2_b"]),
        p["g_W1"], row(p["g_b1"]), p["g_W2"], row(p["g_b2"]),
        p["c_W"], row(p["c_b"]), row(p["c_ln_g"]), row(p["c_ln_b"]),
        p["e_W1"], row(p["e_b1"]), row(p["e_ln1_g"]), row(p["e_ln1_b"]),
        p["e_W2"], row(p["e_b2"]), row(p["e_ln2_g"]), row(p["e_ln2_b"]),
    ]
    in_specs = [
        pl.BlockSpec((B, JB), lambda j: (0, j)),
        pl.BlockSpec((B, JB), lambda j: (0, j)),
        pl.BlockSpec((JB, D), lambda j: (j, 0)),
    ] + [_full(a.shape) for a in in_arrays[3:]]

    mu, lv = pl.pallas_call(
        _body,
        grid=(NJ,),
        in_specs=in_specs,
        out_specs=[_full((B, LATENT)), _full((B, LATENT))],
        out_shape=[jax.ShapeDtypeStruct((B, LATENT), jnp.float32),
                   jax.ShapeDtypeStruct((B, LATENT), jnp.float32)],
        scratch_shapes=[pltpu.VMEM((W * B, D), jnp.float32),
                        pltpu.VMEM((W * B, D), jnp.float32),
                        pltpu.VMEM((R, W * B), jnp.float32),
                        pltpu.VMEM((p["g_W2"].shape[0], W * B), jnp.float32)],
        compiler_params=pltpu.CompilerParams(
            dimension_semantics=("arbitrary",)),
    )(*in_arrays)
    return (mu, lv)
